# Initial kernel scaffold; baseline (speedup 1.0000x reference)
import jax
import jax.numpy as jnp
from jax.experimental import pallas as pl


def kernel(x, edge_index, W1, al1, ar1, b1, g1, be1, W2, al2, ar2, b2, g2, be2, W3, al3, ar3, b3):
    raise NotImplementedError("write your pallas kernel here")


# speedup vs baseline: 21.0612x; 21.0612x over previous
import functools

import jax
import jax.numpy as jnp
from jax import lax
from jax.experimental import pallas as pl
from jax.experimental.pallas import tpu as pltpu
from jax.experimental.pallas import tpu_sc as plsc

N = 10000
E = 160000
IN_DIM = 256
HIDDEN = 256
HEADS = 8
OF = HIDDEN // HEADS
OUT_DIM = 64

BN = 400
GRID = N // BN

ROW = 144
ROW3 = 80
HH = HEADS // 2
NT = 16
RPT = 640
RPT_L = N - (NT - 1) * RPT

EPT = E // NT
CHUNK = 80
NCH = EPT // CHUNK

EPT3 = E // (2 * NT)
CHUNK3 = 40
NCH3 = EPT3 // CHUNK3


def _dense_body(x_ref, st_ref, g_ref, be_ref, w_ref, al_ref, ar_ref,
                tab_ref, er_ref, *, normalize, heads3):
    xb = x_ref[...]
    if normalize:
        mu = st_ref[0, :] * (1.0 / N)
        var = st_ref[1, :] * (1.0 / N) - mu * mu
        sc = g_ref[...] * lax.rsqrt(var + 1e-5)
        xb = (xb - mu) * sc + be_ref[...]
    feat = jnp.dot(xb, w_ref[...], preferred_element_type=jnp.float32)
    el = jnp.dot(feat, al_ref[...], preferred_element_type=jnp.float32)
    er = jnp.dot(feat, ar_ref[...], preferred_element_type=jnp.float32)
    if heads3:
        tab_ref[:, 0:64] = feat
        tab_ref[:, 64:65] = el
        tab_ref[:, 65:80] = jnp.zeros((BN, 15), jnp.float32)
        er_ref[:, 0:1] = er
        er_ref[:, 1:16] = jnp.zeros((BN, 15), jnp.float32)
    else:
        z = jnp.zeros((BN, 12), jnp.float32)
        tab_ref[0, :, 0:128] = feat[:, 0:128]
        tab_ref[0, :, 128:132] = el[:, 0:4]
        tab_ref[0, :, 132:144] = z
        tab_ref[1, :, 0:128] = feat[:, 128:256]
        tab_ref[1, :, 128:132] = el[:, 4:8]
        tab_ref[1, :, 132:144] = z
        er_ref[:, 0:8] = er
        er_ref[:, 8:16] = jnp.zeros((BN, 8), jnp.float32)


def _dense_call(h, stats, g, be, w, al, ar, *, normalize, heads3):
    if heads3:
        outs = (jax.ShapeDtypeStruct((N, ROW3), jnp.float32),
                jax.ShapeDtypeStruct((N, 16), jnp.float32))
        out_specs = (pl.BlockSpec((BN, ROW3), lambda i: (i, 0)),
                     pl.BlockSpec((BN, 16), lambda i: (i, 0)))
        odim = OUT_DIM
    else:
        outs = (jax.ShapeDtypeStruct((2, N, ROW), jnp.float32),
                jax.ShapeDtypeStruct((N, 16), jnp.float32))
        out_specs = (pl.BlockSpec((2, BN, ROW), lambda i: (0, i, 0)),
                     pl.BlockSpec((BN, 16), lambda i: (i, 0)))
        odim = HIDDEN
    nh = 1 if heads3 else HEADS
    return pl.pallas_call(
        functools.partial(_dense_body, normalize=normalize, heads3=heads3),
        grid=(GRID,),
        in_specs=[
            pl.BlockSpec((BN, HIDDEN), lambda i: (i, 0)),
            pl.BlockSpec((2, HIDDEN), lambda i: (0, 0)),
            pl.BlockSpec((HIDDEN,), lambda i: (0,)),
            pl.BlockSpec((HIDDEN,), lambda i: (0,)),
            pl.BlockSpec((HIDDEN, odim), lambda i: (0, 0)),
            pl.BlockSpec((odim, nh), lambda i: (0, 0)),
            pl.BlockSpec((odim, nh), lambda i: (0, 0)),
        ],
        out_specs=out_specs,
        out_shape=outs,
    )(h, stats, g, be, w, al, ar)


def _combine_body(nd_ref, b_ref, erep_ref, h_ref, st_ref):
    i = pl.program_id(0)
    parts = []
    for cc in range(2):
        num = nd_ref[cc, :, 0:128]
        den = jnp.dot(nd_ref[cc, :, 132:136], erep_ref[...],
                      preferred_element_type=jnp.float32)
        ok = den > 0
        parts.append(jnp.where(ok, num / jnp.where(ok, den, 1.0), 0.0))
    hv = jnp.concatenate(parts, axis=1) + b_ref[...]
    h_ref[...] = hv

    @pl.when(i == 0)
    def _():
        st_ref[...] = jnp.zeros((2, HIDDEN), jnp.float32)

    st_ref[0, :] = st_ref[0, :] + jnp.sum(hv, axis=0)
    st_ref[1, :] = st_ref[1, :] + jnp.sum(hv * hv, axis=0)


def _combine_call(nd, b, erep):
    return pl.pallas_call(
        _combine_body,
        grid=(GRID,),
        in_specs=[
            pl.BlockSpec((2, BN, ROW), lambda i: (0, i, 0)),
            pl.BlockSpec((HIDDEN,), lambda i: (0,)),
            pl.BlockSpec((4, 128), lambda i: (0, 0)),
        ],
        out_specs=(pl.BlockSpec((BN, HIDDEN), lambda i: (i, 0)),
                   pl.BlockSpec((2, HIDDEN), lambda i: (0, 0))),
        out_shape=(jax.ShapeDtypeStruct((N, HIDDEN), jnp.float32),
                   jax.ShapeDtypeStruct((2, HIDDEN), jnp.float32)),
    )(nd, b, erep)


def _final_body(nd_ref, b_ref, out_ref):
    num = nd_ref[0, :, 0:64] + nd_ref[1, :, 0:64]
    den = nd_ref[0, :, 65:66] + nd_ref[1, :, 65:66]
    ok = den > 0
    h3 = jnp.where(ok, num / jnp.where(ok, den, 1.0), 0.0) + b_ref[...]
    m = jnp.max(h3, axis=1, keepdims=True)
    r = h3 - m
    lse = jnp.log(jnp.sum(jnp.exp(r), axis=1, keepdims=True))
    out_ref[...] = r - lse


def _final_call(nd3, b3):
    return pl.pallas_call(
        _final_body,
        grid=(GRID,),
        in_specs=[
            pl.BlockSpec((2, BN, ROW3), lambda i: (0, i, 0)),
            pl.BlockSpec((OUT_DIM,), lambda i: (0,)),
        ],
        out_specs=pl.BlockSpec((BN, OUT_DIM), lambda i: (i, 0)),
        out_shape=jax.ShapeDtypeStruct((N, OUT_DIM), jnp.float32),
    )(nd3, b3)


def _full16(v):
    return jnp.full((16,), v, jnp.int32)


def _striped(s, fn):
    @pl.when(s < NT - 1)
    def _():
        fn(s * RPT, RPT)

    @pl.when(s == NT - 1)
    def _():
        fn((NT - 1) * RPT, RPT_L)


def _sc_edge12_body(tab, er, srci, dsti, zer, out, acc, idx_s, idx_d, rows,
                    er_rows, sem, sem2):
    c = lax.axis_index("c")
    s = lax.axis_index("s")
    cn = c * N
    ch = c * HH
    _striped(s, lambda o, n: pltpu.sync_copy(zer.at[pl.ds(0, n)],
                                             acc.at[pl.ds(o, n)]))
    plsc.subcore_barrier()
    base = s * EPT

    def chunk(i, carry):
        off = base + i * CHUNK
        pltpu.sync_copy(srci.at[pl.ds(off, CHUNK)], idx_s)
        pltpu.sync_copy(dsti.at[pl.ds(off, CHUNK)], idx_d)
        for q in range(CHUNK // 16):
            sl = pl.ds(q * 16, 16)
            idx_s[sl] = idx_s[sl] + cn
        cp1 = pltpu.async_copy(tab.at[idx_s], rows, sem)
        cp2 = pltpu.async_copy(er.at[idx_d], er_rows, sem2)
        cp1.wait()
        cp2.wait()
        for g in range(CHUNK // 16):
            e16 = jnp.arange(16, dtype=jnp.int32) + (g * 16)
            for h in range(HH):
                erv = plsc.load_gather(er_rows, [e16, jnp.full((16,), ch + h,
                                                               jnp.int32)])
                elv = plsc.load_gather(rows, [e16, _full16(128 + h)])
                sv = elv + erv
                sv = jnp.maximum(sv, 0.2 * sv)
                w = jnp.exp(sv)
                plsc.store_scatter(rows, [e16, _full16(132 + h)], w)
        for e in range(CHUNK):
            for h in range(HH):
                wsp = plsc.load_gather(rows, [_full16(e), _full16(132 + h)])
                for j in (2 * h, 2 * h + 1):
                    sl = pl.ds(j * 16, 16)
                    rows[e, sl] = rows[e, sl] * wsp
        pltpu.sync_copy(rows, acc.at[idx_d], add=True)
        return carry

    lax.fori_loop(0, NCH, chunk, 0)
    plsc.subcore_barrier()
    _striped(s, lambda o, n: pltpu.sync_copy(acc.at[pl.ds(o, n)],
                                             out.at[pl.ds(cn + o, n)]))


def _sc_edge3_body(tab, er, srci, dsti, zer, out, acc, idx_s, idx_d, rows,
                   er_rows, sem, sem2):
    c = lax.axis_index("c")
    s = lax.axis_index("s")
    _striped(s, lambda o, n: pltpu.sync_copy(zer.at[pl.ds(0, n)],
                                             acc.at[pl.ds(o, n)]))
    plsc.subcore_barrier()
    base = (c * NT + s) * EPT3

    def chunk(i, carry):
        off = base + i * CHUNK3
        pltpu.sync_copy(srci.at[pl.ds(off, CHUNK3)], idx_s)
        pltpu.sync_copy(dsti.at[pl.ds(off, CHUNK3)], idx_d)
        cp1 = pltpu.async_copy(tab.at[idx_s], rows, sem)
        cp2 = pltpu.async_copy(er.at[idx_d], er_rows, sem2)
        cp1.wait()
        cp2.wait()
        for g0 in (0, 16, 24):
            e16 = jnp.arange(16, dtype=jnp.int32) + g0
            erv = plsc.load_gather(er_rows, [e16, _full16(0)])
            elv = plsc.load_gather(rows, [e16, _full16(64)])
            sv = elv + erv
            sv = jnp.maximum(sv, 0.2 * sv)
            w = jnp.exp(sv)
            plsc.store_scatter(rows, [e16, _full16(65)], w)
        for e in range(CHUNK3):
            wsp = plsc.load_gather(rows, [_full16(e), _full16(65)])
            for j in range(4):
                sl = pl.ds(j * 16, 16)
                rows[e, sl] = rows[e, sl] * wsp
        pltpu.sync_copy(rows, acc.at[idx_d], add=True)
        return carry

    lax.fori_loop(0, NCH3, chunk, 0)
    plsc.subcore_barrier()
    _striped(s, lambda o, n: pltpu.sync_copy(acc.at[pl.ds(o, n)],
                                             out.at[pl.ds(c * N + o, n)]))


@functools.lru_cache(maxsize=1)
def _sc_kernels():
    mesh = plsc.VectorSubcoreMesh(core_axis_name="c", subcore_axis_name="s",
                                  num_cores=2)
    cp = pltpu.CompilerParams(needs_layout_passes=False,
                              use_tc_tiling_on_sc=False)
    edge12 = pl.kernel(
        _sc_edge12_body,
        mesh=mesh,
        compiler_params=cp,
        out_type=jax.ShapeDtypeStruct((2 * N, ROW), jnp.float32),
        scratch_types=[
            pltpu.VMEM_SHARED((N, ROW), jnp.float32),
            pltpu.VMEM((CHUNK,), jnp.int32),
            pltpu.VMEM((CHUNK,), jnp.int32),
            pltpu.VMEM((CHUNK, ROW), jnp.float32),
            pltpu.VMEM((CHUNK, 16), jnp.float32),
            pltpu.SemaphoreType.DMA,
            pltpu.SemaphoreType.DMA,
        ],
    )
    edge3 = pl.kernel(
        _sc_edge3_body,
        mesh=mesh,
        compiler_params=cp,
        out_type=jax.ShapeDtypeStruct((2 * N, ROW3), jnp.float32),
        scratch_types=[
            pltpu.VMEM_SHARED((N, ROW3), jnp.float32),
            pltpu.VMEM((CHUNK3,), jnp.int32),
            pltpu.VMEM((CHUNK3,), jnp.int32),
            pltpu.VMEM((CHUNK3, ROW3), jnp.float32),
            pltpu.VMEM((CHUNK3, 16), jnp.float32),
            pltpu.SemaphoreType.DMA,
            pltpu.SemaphoreType.DMA,
        ],
    )
    return edge12, edge3


def _head_proj(a):
    h = a.shape[0]
    eye = jnp.eye(h, dtype=jnp.float32)
    return (a[:, :, None] * eye[:, None, :]).reshape(h * a.shape[1], h)


def kernel(x, edge_index, W1, al1, ar1, b1, g1, be1,
           W2, al2, ar2, b2, g2, be2, W3, al3, ar3, b3):
    src = edge_index[0].astype(jnp.int32)
    dst = edge_index[1].astype(jnp.int32)

    al1p, ar1p = _head_proj(al1), _head_proj(ar1)
    al2p, ar2p = _head_proj(al2), _head_proj(ar2)
    al3p, ar3p = al3.reshape(OUT_DIM, 1), ar3.reshape(OUT_DIM, 1)
    erep = jnp.repeat(jnp.eye(4, dtype=jnp.float32), OF, axis=1)
    zer = jnp.zeros((RPT, ROW), jnp.float32)
    zer3 = jnp.zeros((RPT, ROW3), jnp.float32)
    st0 = jnp.zeros((2, HIDDEN), jnp.float32)
    gd = jnp.ones((HIDDEN,), jnp.float32)

    sc_edge12, sc_edge3 = _sc_kernels()

    tab1, er1 = _dense_call(x, st0, gd, st0[0], W1, al1p, ar1p,
                            normalize=False, heads3=False)
    nd1 = sc_edge12(tab1.reshape(2 * N, ROW), er1, src, dst, zer)
    h1, st1 = _combine_call(nd1.reshape(2, N, ROW), b1, erep)

    tab2, er2 = _dense_call(h1, st1, g1, be1, W2, al2p, ar2p,
                            normalize=True, heads3=False)
    nd2 = sc_edge12(tab2.reshape(2 * N, ROW), er2, src, dst, zer)
    h2, st2 = _combine_call(nd2.reshape(2, N, ROW), b2, erep)

    tab3, er3 = _dense_call(h2, st2, g2, be2, W3, al3p, ar3p,
                            normalize=True, heads3=True)
    nd3 = sc_edge3(tab3, er3, src, dst, zer3)
    return _final_call(nd3.reshape(2, N, ROW3), b3)


# speedup vs baseline: 26.1050x; 1.2395x over previous
import functools

import jax
import jax.numpy as jnp
from jax import lax
from jax.experimental import pallas as pl
from jax.experimental.pallas import tpu as pltpu
from jax.experimental.pallas import tpu_sc as plsc

N = 10000
E = 160000
IN_DIM = 256
HIDDEN = 256
HEADS = 8
OF = HIDDEN // HEADS
OUT_DIM = 64

BN = 400
GRID = N // BN

ROW = 144
ROW3 = 80
HH = HEADS // 2
NT = 16
RPT = 640
RPT_L = N - (NT - 1) * RPT

EPT = E // NT
CHUNK = 40
NCH = EPT // CHUNK

EPT3 = E // (2 * NT)
CHUNK3 = 40
NCH3 = EPT3 // CHUNK3


def _dense_body(x_ref, st_ref, g_ref, be_ref, w_ref, al_ref, ar_ref,
                tab_ref, er_ref, *, normalize, heads3):
    xb = x_ref[...]
    if normalize:
        mu = st_ref[0, :] * (1.0 / N)
        var = st_ref[1, :] * (1.0 / N) - mu * mu
        sc = g_ref[...] * lax.rsqrt(var + 1e-5)
        xb = (xb - mu) * sc + be_ref[...]
    feat = jnp.dot(xb, w_ref[...], preferred_element_type=jnp.float32)
    el = jnp.dot(feat, al_ref[...], preferred_element_type=jnp.float32)
    er = jnp.dot(feat, ar_ref[...], preferred_element_type=jnp.float32)
    if heads3:
        tab_ref[:, 0:64] = feat
        tab_ref[:, 64:65] = el
        tab_ref[:, 65:80] = jnp.zeros((BN, 15), jnp.float32)
        er_ref[:, 0:1] = er
        er_ref[:, 1:16] = jnp.zeros((BN, 15), jnp.float32)
    else:
        z = jnp.zeros((BN, 12), jnp.float32)
        tab_ref[0, :, 0:128] = feat[:, 0:128]
        tab_ref[0, :, 128:132] = el[:, 0:4]
        tab_ref[0, :, 132:144] = z
        tab_ref[1, :, 0:128] = feat[:, 128:256]
        tab_ref[1, :, 128:132] = el[:, 4:8]
        tab_ref[1, :, 132:144] = z
        er_ref[:, 0:8] = er
        er_ref[:, 8:16] = jnp.zeros((BN, 8), jnp.float32)


def _dense_call(h, stats, g, be, w, al, ar, *, normalize, heads3):
    if heads3:
        outs = (jax.ShapeDtypeStruct((N, ROW3), jnp.float32),
                jax.ShapeDtypeStruct((N, 16), jnp.float32))
        out_specs = (pl.BlockSpec((BN, ROW3), lambda i: (i, 0)),
                     pl.BlockSpec((BN, 16), lambda i: (i, 0)))
        odim = OUT_DIM
    else:
        outs = (jax.ShapeDtypeStruct((2, N, ROW), jnp.float32),
                jax.ShapeDtypeStruct((N, 16), jnp.float32))
        out_specs = (pl.BlockSpec((2, BN, ROW), lambda i: (0, i, 0)),
                     pl.BlockSpec((BN, 16), lambda i: (i, 0)))
        odim = HIDDEN
    nh = 1 if heads3 else HEADS
    return pl.pallas_call(
        functools.partial(_dense_body, normalize=normalize, heads3=heads3),
        grid=(GRID,),
        in_specs=[
            pl.BlockSpec((BN, HIDDEN), lambda i: (i, 0)),
            pl.BlockSpec((2, HIDDEN), lambda i: (0, 0)),
            pl.BlockSpec((HIDDEN,), lambda i: (0,)),
            pl.BlockSpec((HIDDEN,), lambda i: (0,)),
            pl.BlockSpec((HIDDEN, odim), lambda i: (0, 0)),
            pl.BlockSpec((odim, nh), lambda i: (0, 0)),
            pl.BlockSpec((odim, nh), lambda i: (0, 0)),
        ],
        out_specs=out_specs,
        out_shape=outs,
    )(h, stats, g, be, w, al, ar)


def _combine_body(nd_ref, b_ref, erep_ref, h_ref, st_ref):
    i = pl.program_id(0)
    parts = []
    for cc in range(2):
        num = nd_ref[cc, :, 0:128]
        den = jnp.dot(nd_ref[cc, :, 132:136], erep_ref[...],
                      preferred_element_type=jnp.float32)
        ok = den > 0
        parts.append(jnp.where(ok, num / jnp.where(ok, den, 1.0), 0.0))
    hv = jnp.concatenate(parts, axis=1) + b_ref[...]
    h_ref[...] = hv

    @pl.when(i == 0)
    def _():
        st_ref[...] = jnp.zeros((2, HIDDEN), jnp.float32)

    st_ref[0, :] = st_ref[0, :] + jnp.sum(hv, axis=0)
    st_ref[1, :] = st_ref[1, :] + jnp.sum(hv * hv, axis=0)


def _combine_call(nd, b, erep):
    return pl.pallas_call(
        _combine_body,
        grid=(GRID,),
        in_specs=[
            pl.BlockSpec((2, BN, ROW), lambda i: (0, i, 0)),
            pl.BlockSpec((HIDDEN,), lambda i: (0,)),
            pl.BlockSpec((4, 128), lambda i: (0, 0)),
        ],
        out_specs=(pl.BlockSpec((BN, HIDDEN), lambda i: (i, 0)),
                   pl.BlockSpec((2, HIDDEN), lambda i: (0, 0))),
        out_shape=(jax.ShapeDtypeStruct((N, HIDDEN), jnp.float32),
                   jax.ShapeDtypeStruct((2, HIDDEN), jnp.float32)),
    )(nd, b, erep)


def _final_body(nd_ref, b_ref, out_ref):
    num = nd_ref[0, :, 0:64] + nd_ref[1, :, 0:64]
    den = nd_ref[0, :, 65:66] + nd_ref[1, :, 65:66]
    ok = den > 0
    h3 = jnp.where(ok, num / jnp.where(ok, den, 1.0), 0.0) + b_ref[...]
    m = jnp.max(h3, axis=1, keepdims=True)
    r = h3 - m
    lse = jnp.log(jnp.sum(jnp.exp(r), axis=1, keepdims=True))
    out_ref[...] = r - lse


def _final_call(nd3, b3):
    return pl.pallas_call(
        _final_body,
        grid=(GRID,),
        in_specs=[
            pl.BlockSpec((2, BN, ROW3), lambda i: (0, i, 0)),
            pl.BlockSpec((OUT_DIM,), lambda i: (0,)),
        ],
        out_specs=pl.BlockSpec((BN, OUT_DIM), lambda i: (i, 0)),
        out_shape=jax.ShapeDtypeStruct((N, OUT_DIM), jnp.float32),
    )(nd3, b3)


def _full16(v):
    return jnp.full((16,), v, jnp.int32)


def _striped(s, fn):
    @pl.when(s < NT - 1)
    def _():
        fn(s * RPT, RPT)

    @pl.when(s == NT - 1)
    def _():
        fn((NT - 1) * RPT, RPT_L)


def _sc_edge12_body(tab, er, ei, zer, out, acc,
                    idx0, idx1, idx2, rw0, rw1, rw2, ew0, ew1, ew2,
                    semg, seme, sems):
    c = lax.axis_index("c")
    s = lax.axis_index("s")
    cn = c * N
    ch = c * HH
    idx = (idx0, idx1, idx2)
    rows = (rw0, rw1, rw2)
    errw = (ew0, ew1, ew2)
    _striped(s, lambda o, n: pltpu.sync_copy(zer.at[pl.ds(0, n)],
                                             acc.at[pl.ds(o, n)]))
    plsc.subcore_barrier()
    base = s * EPT

    def fetch(b, i):
        off = base + i * CHUNK
        pltpu.sync_copy(ei.at[:, pl.ds(off, CHUNK)], idx[b])
        for q in range(2):
            sl = pl.ds(q * 16, 16)
            idx[b][0, sl] = idx[b][0, sl] + cn
        sl = pl.ds(24, 16)
        lane = jnp.arange(16, dtype=jnp.int32)
        idx[b][0, sl] = idx[b][0, sl] + jnp.where(lane >= 8, cn, 0)
        pltpu.async_copy(tab.at[idx[b].at[0]], rows[b], semg.at[b])
        pltpu.async_copy(er.at[idx[b].at[1]], errw[b], seme.at[b])

    def compute_scatter(b):
        pltpu.make_async_copy(tab.at[idx[b].at[0]], rows[b], semg.at[b]).wait()
        pltpu.make_async_copy(er.at[idx[b].at[1]], errw[b], seme.at[b]).wait()
        for g0 in (0, 16, 24):
            e16 = jnp.arange(16, dtype=jnp.int32) + g0
            for h in range(HH):
                erv = plsc.load_gather(errw[b], [e16, jnp.full((16,), ch + h,
                                                               jnp.int32)])
                elv = plsc.load_gather(rows[b], [e16, _full16(128 + h)])
                sv = elv + erv
                sv = jnp.maximum(sv, 0.2 * sv)
                w = jnp.exp(sv)
                plsc.store_scatter(rows[b], [e16, _full16(132 + h)], w)
        for e in range(CHUNK):
            for h in range(HH):
                wsp = plsc.load_gather(rows[b], [_full16(e), _full16(132 + h)])
                for j in (2 * h, 2 * h + 1):
                    sl = pl.ds(j * 16, 16)
                    rows[b][e, sl] = rows[b][e, sl] * wsp
        pltpu.async_copy(rows[b], acc.at[idx[b].at[1]], sems.at[b], add=True)

    def wait_scatter(b):
        pltpu.make_async_copy(rows[b], acc.at[idx[b].at[1]], sems.at[b]).wait()

    fetch(0, 0)
    fetch(1, 1)

    def outer(o, carry):
        for b in range(3):
            i = 3 * o + b
            compute_scatter(b)
            b2 = (b + 2) % 3

            @pl.when(i + 2 < NCH)
            def _():
                @pl.when(i >= 1)
                def _():
                    wait_scatter(b2)
                fetch(b2, i + 2)
        return carry

    lax.fori_loop(0, NCH // 3, outer, 0)
    compute_scatter((NCH - 1) % 3)
    wait_scatter(1)
    wait_scatter(2)
    wait_scatter(0)
    plsc.subcore_barrier()
    _striped(s, lambda o, n: pltpu.sync_copy(acc.at[pl.ds(o, n)],
                                             out.at[pl.ds(cn + o, n)]))


def _sc_edge3_body(tab, er, ei, zer, out, acc,
                   idx0, idx1, idx2, rw0, rw1, rw2, ew0, ew1, ew2,
                   semg, seme, sems):
    c = lax.axis_index("c")
    s = lax.axis_index("s")
    idx = (idx0, idx1, idx2)
    rows = (rw0, rw1, rw2)
    errw = (ew0, ew1, ew2)
    _striped(s, lambda o, n: pltpu.sync_copy(zer.at[pl.ds(0, n)],
                                             acc.at[pl.ds(o, n)]))
    plsc.subcore_barrier()
    base = (c * NT + s) * EPT3

    def fetch(b, i):
        off = base + i * CHUNK3
        pltpu.sync_copy(ei.at[:, pl.ds(off, CHUNK3)], idx[b])
        pltpu.async_copy(tab.at[idx[b].at[0]], rows[b], semg.at[b])
        pltpu.async_copy(er.at[idx[b].at[1]], errw[b], seme.at[b])

    def compute_scatter(b):
        pltpu.make_async_copy(tab.at[idx[b].at[0]], rows[b], semg.at[b]).wait()
        pltpu.make_async_copy(er.at[idx[b].at[1]], errw[b], seme.at[b]).wait()
        for g0 in (0, 16, 24):
            e16 = jnp.arange(16, dtype=jnp.int32) + g0
            erv = plsc.load_gather(errw[b], [e16, _full16(0)])
            elv = plsc.load_gather(rows[b], [e16, _full16(64)])
            sv = elv + erv
            sv = jnp.maximum(sv, 0.2 * sv)
            w = jnp.exp(sv)
            plsc.store_scatter(rows[b], [e16, _full16(65)], w)
        for e in range(CHUNK3):
            wsp = plsc.load_gather(rows[b], [_full16(e), _full16(65)])
            for j in range(4):
                sl = pl.ds(j * 16, 16)
                rows[b][e, sl] = rows[b][e, sl] * wsp
        pltpu.async_copy(rows[b], acc.at[idx[b].at[1]], sems.at[b], add=True)

    def wait_scatter(b):
        pltpu.make_async_copy(rows[b], acc.at[idx[b].at[1]], sems.at[b]).wait()

    fetch(0, 0)
    fetch(1, 1)

    def outer(o, carry):
        for b in range(3):
            i = 3 * o + b
            compute_scatter(b)
            b2 = (b + 2) % 3

            @pl.when(i + 2 < NCH3)
            def _():
                @pl.when(i >= 1)
                def _():
                    wait_scatter(b2)
                fetch(b2, i + 2)
        return carry

    lax.fori_loop(0, NCH3 // 3, outer, 0)
    compute_scatter(0)
    compute_scatter(1)
    wait_scatter(2)
    wait_scatter(0)
    wait_scatter(1)
    plsc.subcore_barrier()
    _striped(s, lambda o, n: pltpu.sync_copy(acc.at[pl.ds(o, n)],
                                             out.at[pl.ds(c * N + o, n)]))


@functools.lru_cache(maxsize=1)
def _sc_kernels():
    mesh = plsc.VectorSubcoreMesh(core_axis_name="c", subcore_axis_name="s",
                                  num_cores=2)
    cp = pltpu.CompilerParams(needs_layout_passes=False,
                              use_tc_tiling_on_sc=False)
    edge12 = pl.kernel(
        _sc_edge12_body,
        mesh=mesh,
        compiler_params=cp,
        out_type=jax.ShapeDtypeStruct((2 * N, ROW), jnp.float32),
        scratch_types=(
            [pltpu.VMEM_SHARED((N, ROW), jnp.float32)]
            + [pltpu.VMEM((2, CHUNK), jnp.int32)] * 3
            + [pltpu.VMEM((CHUNK, ROW), jnp.float32)] * 3
            + [pltpu.VMEM((CHUNK, 16), jnp.float32)] * 3
            + [pltpu.SemaphoreType.DMA((3,))] * 3
        ),
    )
    edge3 = pl.kernel(
        _sc_edge3_body,
        mesh=mesh,
        compiler_params=cp,
        out_type=jax.ShapeDtypeStruct((2 * N, ROW3), jnp.float32),
        scratch_types=(
            [pltpu.VMEM_SHARED((N, ROW3), jnp.float32)]
            + [pltpu.VMEM((2, CHUNK3), jnp.int32)] * 3
            + [pltpu.VMEM((CHUNK3, ROW3), jnp.float32)] * 3
            + [pltpu.VMEM((CHUNK3, 16), jnp.float32)] * 3
            + [pltpu.SemaphoreType.DMA((3,))] * 3
        ),
    )
    return edge12, edge3


def _head_proj(a):
    h = a.shape[0]
    eye = jnp.eye(h, dtype=jnp.float32)
    return (a[:, :, None] * eye[:, None, :]).reshape(h * a.shape[1], h)


def kernel(x, edge_index, W1, al1, ar1, b1, g1, be1,
           W2, al2, ar2, b2, g2, be2, W3, al3, ar3, b3):
    ei = edge_index.astype(jnp.int32)

    al1p, ar1p = _head_proj(al1), _head_proj(ar1)
    al2p, ar2p = _head_proj(al2), _head_proj(ar2)
    al3p, ar3p = al3.reshape(OUT_DIM, 1), ar3.reshape(OUT_DIM, 1)
    erep = jnp.repeat(jnp.eye(4, dtype=jnp.float32), OF, axis=1)
    zer = jnp.zeros((RPT, ROW), jnp.float32)
    zer3 = jnp.zeros((RPT, ROW3), jnp.float32)
    st0 = jnp.zeros((2, HIDDEN), jnp.float32)
    gd = jnp.ones((HIDDEN,), jnp.float32)

    sc_edge12, sc_edge3 = _sc_kernels()

    tab1, er1 = _dense_call(x, st0, gd, st0[0], W1, al1p, ar1p,
                            normalize=False, heads3=False)
    nd1 = sc_edge12(tab1.reshape(2 * N, ROW), er1, ei, zer)
    h1, st1 = _combine_call(nd1.reshape(2, N, ROW), b1, erep)

    tab2, er2 = _dense_call(h1, st1, g1, be1, W2, al2p, ar2p,
                            normalize=True, heads3=False)
    nd2 = sc_edge12(tab2.reshape(2 * N, ROW), er2, ei, zer)
    h2, st2 = _combine_call(nd2.reshape(2, N, ROW), b2, erep)

    tab3, er3 = _dense_call(h2, st2, g2, be2, W3, al3p, ar3p,
                            normalize=True, heads3=True)
    nd3 = sc_edge3(tab3, er3, ei, zer3)
    return _final_call(nd3.reshape(2, N, ROW3), b3)


# speedup vs baseline: 39.3427x; 1.5071x over previous
import functools

import jax
import jax.numpy as jnp
from jax import lax
from jax.experimental import pallas as pl
from jax.experimental.pallas import tpu as pltpu
from jax.experimental.pallas import tpu_sc as plsc

N = 10000
E = 160000
IN_DIM = 256
HIDDEN = 256
HEADS = 8
OF = HIDDEN // HEADS
OUT_DIM = 64

BN = 400
GRID = N // BN

ROW = 144
ROW3 = 80
HH = HEADS // 2
NT = 16
RPT = 640
RPT_L = N - (NT - 1) * RPT

EPT = E // NT
CHUNK = 80
NCH = EPT // CHUNK

EPT3 = E // (2 * NT)
CHUNK3 = 40
NCH3 = EPT3 // CHUNK3


def _dense_body(x_ref, st_ref, g_ref, be_ref, w_ref, al_ref, ar_ref,
                tab_ref, er_ref, *, normalize, heads3):
    xb = x_ref[...]
    if normalize:
        mu = st_ref[0, :] * (1.0 / N)
        var = st_ref[1, :] * (1.0 / N) - mu * mu
        sc = g_ref[...] * lax.rsqrt(var + 1e-5)
        xb = (xb - mu) * sc + be_ref[...]
    feat = jnp.dot(xb, w_ref[...], preferred_element_type=jnp.float32)
    el = jnp.dot(feat, al_ref[...], preferred_element_type=jnp.float32)
    er = jnp.dot(feat, ar_ref[...], preferred_element_type=jnp.float32)
    if heads3:
        tab_ref[:, 0:64] = feat
        tab_ref[:, 64:65] = el
        tab_ref[:, 65:80] = jnp.zeros((BN, 15), jnp.float32)
        er_ref[:, 0:1] = er
        er_ref[:, 1:16] = jnp.zeros((BN, 15), jnp.float32)
    else:
        z = jnp.zeros((BN, 12), jnp.float32)
        tab_ref[0, :, 0:128] = feat[:, 0:128]
        tab_ref[0, :, 128:132] = el[:, 0:4]
        tab_ref[0, :, 132:144] = z
        tab_ref[1, :, 0:128] = feat[:, 128:256]
        tab_ref[1, :, 128:132] = el[:, 4:8]
        tab_ref[1, :, 132:144] = z
        er_ref[:, 0:8] = er
        er_ref[:, 8:16] = jnp.zeros((BN, 8), jnp.float32)


def _dense_call(h, stats, g, be, w, al, ar, *, normalize, heads3):
    if heads3:
        outs = (jax.ShapeDtypeStruct((N, ROW3), jnp.float32),
                jax.ShapeDtypeStruct((N, 16), jnp.float32))
        out_specs = (pl.BlockSpec((BN, ROW3), lambda i: (i, 0)),
                     pl.BlockSpec((BN, 16), lambda i: (i, 0)))
        odim = OUT_DIM
    else:
        outs = (jax.ShapeDtypeStruct((2, N, ROW), jnp.float32),
                jax.ShapeDtypeStruct((N, 16), jnp.float32))
        out_specs = (pl.BlockSpec((2, BN, ROW), lambda i: (0, i, 0)),
                     pl.BlockSpec((BN, 16), lambda i: (i, 0)))
        odim = HIDDEN
    nh = 1 if heads3 else HEADS
    return pl.pallas_call(
        functools.partial(_dense_body, normalize=normalize, heads3=heads3),
        grid=(GRID,),
        in_specs=[
            pl.BlockSpec((BN, HIDDEN), lambda i: (i, 0)),
            pl.BlockSpec((2, HIDDEN), lambda i: (0, 0)),
            pl.BlockSpec((HIDDEN,), lambda i: (0,)),
            pl.BlockSpec((HIDDEN,), lambda i: (0,)),
            pl.BlockSpec((HIDDEN, odim), lambda i: (0, 0)),
            pl.BlockSpec((odim, nh), lambda i: (0, 0)),
            pl.BlockSpec((odim, nh), lambda i: (0, 0)),
        ],
        out_specs=out_specs,
        out_shape=outs,
    )(h, stats, g, be, w, al, ar)


def _combine_body(nd_ref, b_ref, erep_ref, h_ref, st_ref):
    i = pl.program_id(0)
    parts = []
    for cc in range(2):
        num = nd_ref[cc, :, 0:128]
        den = jnp.dot(nd_ref[cc, :, 132:136], erep_ref[...],
                      preferred_element_type=jnp.float32)
        ok = den > 0
        parts.append(jnp.where(ok, num / jnp.where(ok, den, 1.0), 0.0))
    hv = jnp.concatenate(parts, axis=1) + b_ref[...]
    h_ref[...] = hv

    @pl.when(i == 0)
    def _():
        st_ref[...] = jnp.zeros((2, HIDDEN), jnp.float32)

    st_ref[0, :] = st_ref[0, :] + jnp.sum(hv, axis=0)
    st_ref[1, :] = st_ref[1, :] + jnp.sum(hv * hv, axis=0)


def _combine_call(nd, b, erep):
    return pl.pallas_call(
        _combine_body,
        grid=(GRID,),
        in_specs=[
            pl.BlockSpec((2, BN, ROW), lambda i: (0, i, 0)),
            pl.BlockSpec((HIDDEN,), lambda i: (0,)),
            pl.BlockSpec((4, 128), lambda i: (0, 0)),
        ],
        out_specs=(pl.BlockSpec((BN, HIDDEN), lambda i: (i, 0)),
                   pl.BlockSpec((2, HIDDEN), lambda i: (0, 0))),
        out_shape=(jax.ShapeDtypeStruct((N, HIDDEN), jnp.float32),
                   jax.ShapeDtypeStruct((2, HIDDEN), jnp.float32)),
    )(nd, b, erep)


def _final_body(nd_ref, b_ref, out_ref):
    num = nd_ref[0, :, 0:64] + nd_ref[1, :, 0:64]
    den = nd_ref[0, :, 65:66] + nd_ref[1, :, 65:66]
    ok = den > 0
    h3 = jnp.where(ok, num / jnp.where(ok, den, 1.0), 0.0) + b_ref[...]
    m = jnp.max(h3, axis=1, keepdims=True)
    r = h3 - m
    lse = jnp.log(jnp.sum(jnp.exp(r), axis=1, keepdims=True))
    out_ref[...] = r - lse


def _final_call(nd3, b3):
    return pl.pallas_call(
        _final_body,
        grid=(GRID,),
        in_specs=[
            pl.BlockSpec((2, BN, ROW3), lambda i: (0, i, 0)),
            pl.BlockSpec((OUT_DIM,), lambda i: (0,)),
        ],
        out_specs=pl.BlockSpec((BN, OUT_DIM), lambda i: (i, 0)),
        out_shape=jax.ShapeDtypeStruct((N, OUT_DIM), jnp.float32),
    )(nd3, b3)


def _full16(v):
    return jnp.full((16,), v, jnp.int32)


def _striped(s, fn):
    @pl.when(s < NT - 1)
    def _():
        fn(s * RPT, RPT)

    @pl.when(s == NT - 1)
    def _():
        fn((NT - 1) * RPT, RPT_L)


def _sc_edge12_body(tab, er, ei, zer, out, acc,
                    idx0, idx1, idx2, rw0, rw1, rw2, ew0, ew1, ew2,
                    semg, seme, sems):
    c = lax.axis_index("c")
    s = lax.axis_index("s")
    cn = c * N
    ch = c * HH
    idx = (idx0, idx1, idx2)
    rows = (rw0, rw1, rw2)
    errw = (ew0, ew1, ew2)
    _striped(s, lambda o, n: pltpu.sync_copy(zer.at[pl.ds(0, n)],
                                             acc.at[pl.ds(o, n)]))
    plsc.subcore_barrier()
    base = s * EPT

    def fetch(b, i):
        off = base + i * CHUNK
        pltpu.sync_copy(ei.at[:, pl.ds(off, CHUNK)], idx[b])
        for q in range(CHUNK // 16):
            sl = pl.ds(q * 16, 16)
            idx[b][0, sl] = idx[b][0, sl] + cn
        pltpu.async_copy(tab.at[idx[b].at[0]], rows[b], semg.at[b])
        pltpu.async_copy(er.at[idx[b].at[1]], errw[b], seme.at[b])

    def compute_scatter(b):
        pltpu.make_async_copy(tab.at[idx[b].at[0]], rows[b], semg.at[b]).wait()
        pltpu.make_async_copy(er.at[idx[b].at[1]], errw[b], seme.at[b]).wait()
        for g in range(CHUNK // 16):
            e16 = jnp.arange(16, dtype=jnp.int32) + g * 16
            for h in range(HH):
                erv = plsc.load_gather(errw[b], [e16, jnp.full((16,), ch + h,
                                                               jnp.int32)])
                elv = plsc.load_gather(rows[b], [e16, _full16(128 + h)])
                sv = elv + erv
                sv = jnp.maximum(sv, 0.2 * sv)
                w = jnp.exp(sv)
                plsc.store_scatter(rows[b], [e16, _full16(132 + h)], w)
        def escale(e0, carry):
            for k in range(8):
                e = e0 * 8 + k
                for h in range(HH):
                    wsp = plsc.load_gather(rows[b], [jnp.full((16,), e,
                                                              jnp.int32),
                                                     _full16(132 + h)])
                    for j in (2 * h, 2 * h + 1):
                        sl = pl.ds(j * 16, 16)
                        rows[b][e, sl] = rows[b][e, sl] * wsp
            return carry

        lax.fori_loop(0, CHUNK // 8, escale, 0)
        pltpu.async_copy(rows[b], acc.at[idx[b].at[1]], sems.at[b], add=True)

    def wait_scatter(b):
        pltpu.make_async_copy(rows[b], acc.at[idx[b].at[1]], sems.at[b]).wait()

    fetch(0, 0)
    fetch(1, 1)

    def outer(o, carry):
        for b in range(3):
            i = 3 * o + b
            compute_scatter(b)
            b2 = (b + 2) % 3

            @pl.when(i + 2 < NCH)
            def _():
                @pl.when(i >= 1)
                def _():
                    wait_scatter(b2)
                fetch(b2, i + 2)
        return carry

    lax.fori_loop(0, NCH // 3, outer, 0)
    compute_scatter(0)
    compute_scatter(1)
    wait_scatter(2)
    wait_scatter(0)
    wait_scatter(1)
    plsc.subcore_barrier()
    _striped(s, lambda o, n: pltpu.sync_copy(acc.at[pl.ds(o, n)],
                                             out.at[pl.ds(cn + o, n)]))


def _sc_edge3_body(tab, er, ei, zer, out, acc,
                   idx0, idx1, idx2, rw0, rw1, rw2, ew0, ew1, ew2,
                   semg, seme, sems):
    c = lax.axis_index("c")
    s = lax.axis_index("s")
    idx = (idx0, idx1, idx2)
    rows = (rw0, rw1, rw2)
    errw = (ew0, ew1, ew2)
    _striped(s, lambda o, n: pltpu.sync_copy(zer.at[pl.ds(0, n)],
                                             acc.at[pl.ds(o, n)]))
    plsc.subcore_barrier()
    base = (c * NT + s) * EPT3

    def fetch(b, i):
        off = base + i * CHUNK3
        pltpu.sync_copy(ei.at[:, pl.ds(off, CHUNK3)], idx[b])
        pltpu.async_copy(tab.at[idx[b].at[0]], rows[b], semg.at[b])
        pltpu.async_copy(er.at[idx[b].at[1]], errw[b], seme.at[b])

    def compute_scatter(b):
        pltpu.make_async_copy(tab.at[idx[b].at[0]], rows[b], semg.at[b]).wait()
        pltpu.make_async_copy(er.at[idx[b].at[1]], errw[b], seme.at[b]).wait()
        for g0 in (0, 16, 24):
            e16 = jnp.arange(16, dtype=jnp.int32) + g0
            erv = plsc.load_gather(errw[b], [e16, _full16(0)])
            elv = plsc.load_gather(rows[b], [e16, _full16(64)])
            sv = elv + erv
            sv = jnp.maximum(sv, 0.2 * sv)
            w = jnp.exp(sv)
            plsc.store_scatter(rows[b], [e16, _full16(65)], w)
        def escale(e0, carry):
            for k in range(8):
                e = e0 * 8 + k
                wsp = plsc.load_gather(rows[b], [jnp.full((16,), e, jnp.int32),
                                                 _full16(65)])
                for j in range(4):
                    sl = pl.ds(j * 16, 16)
                    rows[b][e, sl] = rows[b][e, sl] * wsp
            return carry

        lax.fori_loop(0, CHUNK3 // 8, escale, 0)
        pltpu.async_copy(rows[b], acc.at[idx[b].at[1]], sems.at[b], add=True)

    def wait_scatter(b):
        pltpu.make_async_copy(rows[b], acc.at[idx[b].at[1]], sems.at[b]).wait()

    fetch(0, 0)
    fetch(1, 1)

    def outer(o, carry):
        for b in range(3):
            i = 3 * o + b
            compute_scatter(b)
            b2 = (b + 2) % 3

            @pl.when(i + 2 < NCH3)
            def _():
                @pl.when(i >= 1)
                def _():
                    wait_scatter(b2)
                fetch(b2, i + 2)
        return carry

    lax.fori_loop(0, NCH3 // 3, outer, 0)
    compute_scatter(0)
    compute_scatter(1)
    wait_scatter(2)
    wait_scatter(0)
    wait_scatter(1)
    plsc.subcore_barrier()
    _striped(s, lambda o, n: pltpu.sync_copy(acc.at[pl.ds(o, n)],
                                             out.at[pl.ds(c * N + o, n)]))


@functools.lru_cache(maxsize=1)
def _sc_kernels():
    mesh = plsc.VectorSubcoreMesh(core_axis_name="c", subcore_axis_name="s",
                                  num_cores=2)
    cp = pltpu.CompilerParams(needs_layout_passes=False,
                              use_tc_tiling_on_sc=False)
    edge12 = pl.kernel(
        _sc_edge12_body,
        mesh=mesh,
        compiler_params=cp,
        out_type=jax.ShapeDtypeStruct((2 * N, ROW), jnp.float32),
        scratch_types=(
            [pltpu.VMEM_SHARED((N, ROW), jnp.float32)]
            + [pltpu.VMEM((2, CHUNK), jnp.int32)] * 3
            + [pltpu.VMEM((CHUNK, ROW), jnp.float32)] * 3
            + [pltpu.VMEM((CHUNK, 16), jnp.float32)] * 3
            + [pltpu.SemaphoreType.DMA((3,))] * 3
        ),
    )
    edge3 = pl.kernel(
        _sc_edge3_body,
        mesh=mesh,
        compiler_params=cp,
        out_type=jax.ShapeDtypeStruct((2 * N, ROW3), jnp.float32),
        scratch_types=(
            [pltpu.VMEM_SHARED((N, ROW3), jnp.float32)]
            + [pltpu.VMEM((2, CHUNK3), jnp.int32)] * 3
            + [pltpu.VMEM((CHUNK3, ROW3), jnp.float32)] * 3
            + [pltpu.VMEM((CHUNK3, 16), jnp.float32)] * 3
            + [pltpu.SemaphoreType.DMA((3,))] * 3
        ),
    )
    return edge12, edge3


def _head_proj(a):
    h = a.shape[0]
    eye = jnp.eye(h, dtype=jnp.float32)
    return (a[:, :, None] * eye[:, None, :]).reshape(h * a.shape[1], h)


def kernel(x, edge_index, W1, al1, ar1, b1, g1, be1,
           W2, al2, ar2, b2, g2, be2, W3, al3, ar3, b3):
    ei = edge_index.astype(jnp.int32)

    al1p, ar1p = _head_proj(al1), _head_proj(ar1)
    al2p, ar2p = _head_proj(al2), _head_proj(ar2)
    al3p, ar3p = al3.reshape(OUT_DIM, 1), ar3.reshape(OUT_DIM, 1)
    erep = jnp.repeat(jnp.eye(4, dtype=jnp.float32), OF, axis=1)
    zer = jnp.zeros((RPT, ROW), jnp.float32)
    zer3 = jnp.zeros((RPT, ROW3), jnp.float32)
    st0 = jnp.zeros((2, HIDDEN), jnp.float32)
    gd = jnp.ones((HIDDEN,), jnp.float32)

    sc_edge12, sc_edge3 = _sc_kernels()

    tab1, er1 = _dense_call(x, st0, gd, st0[0], W1, al1p, ar1p,
                            normalize=False, heads3=False)
    nd1 = sc_edge12(tab1.reshape(2 * N, ROW), er1, ei, zer)
    h1, st1 = _combine_call(nd1.reshape(2, N, ROW), b1, erep)

    tab2, er2 = _dense_call(h1, st1, g1, be1, W2, al2p, ar2p,
                            normalize=True, heads3=False)
    nd2 = sc_edge12(tab2.reshape(2 * N, ROW), er2, ei, zer)
    h2, st2 = _combine_call(nd2.reshape(2, N, ROW), b2, erep)

    tab3, er3 = _dense_call(h2, st2, g2, be2, W3, al3p, ar3p,
                            normalize=True, heads3=True)
    nd3 = sc_edge3(tab3, er3, ei, zer3)
    return _final_call(nd3.reshape(2, N, ROW3), b3)


# speedup vs baseline: 58.4044x; 1.4845x over previous
import functools

import jax
import jax.numpy as jnp
from jax import lax
from jax.experimental import pallas as pl
from jax.experimental.pallas import tpu as pltpu
from jax.experimental.pallas import tpu_sc as plsc

N = 10000
E = 160000
IN_DIM = 256
HIDDEN = 256
HEADS = 8
OF = HIDDEN // HEADS
OUT_DIM = 64

BN = 400
GRID = N // BN

ROW = 144
ROW3 = 80
HH = HEADS // 2
NT = 16
RPT = 640
RPT_L = N - (NT - 1) * RPT

EPT = E // NT
CHUNK = 80
NCH = EPT // CHUNK

EPT3 = E // (2 * NT)
CHUNK3 = 40
NCH3 = EPT3 // CHUNK3


def _dense_body(x_ref, st_ref, g_ref, be_ref, w_ref, al_ref, ar_ref,
                tab_ref, er_ref, *, normalize, heads3):
    xb = x_ref[...]
    if normalize:
        mu = st_ref[0, :] * (1.0 / N)
        var = st_ref[1, :] * (1.0 / N) - mu * mu
        sc = g_ref[...] * lax.rsqrt(var + 1e-5)
        xb = (xb - mu) * sc + be_ref[...]
    feat = jnp.dot(xb, w_ref[...], preferred_element_type=jnp.float32)
    el = jnp.dot(feat, al_ref[...], preferred_element_type=jnp.float32)
    er = jnp.dot(feat, ar_ref[...], preferred_element_type=jnp.float32)
    if heads3:
        tab_ref[:, 0:64] = feat
        tab_ref[:, 64:65] = el
        tab_ref[:, 65:80] = jnp.zeros((BN, 15), jnp.float32)
        er_ref[:, 0:1] = er
        er_ref[:, 1:16] = jnp.zeros((BN, 15), jnp.float32)
    else:
        z = jnp.zeros((BN, 12), jnp.float32)
        tab_ref[0, :, 0:128] = feat[:, 0:128]
        tab_ref[0, :, 128:132] = el[:, 0:4]
        tab_ref[0, :, 132:144] = z
        tab_ref[1, :, 0:128] = feat[:, 128:256]
        tab_ref[1, :, 128:132] = el[:, 4:8]
        tab_ref[1, :, 132:144] = z
        er_ref[:, 0:8] = er
        er_ref[:, 8:16] = jnp.zeros((BN, 8), jnp.float32)


def _dense_call(h, stats, g, be, w, al, ar, *, normalize, heads3):
    if heads3:
        outs = (jax.ShapeDtypeStruct((N, ROW3), jnp.float32),
                jax.ShapeDtypeStruct((N, 16), jnp.float32))
        out_specs = (pl.BlockSpec((BN, ROW3), lambda i: (i, 0)),
                     pl.BlockSpec((BN, 16), lambda i: (i, 0)))
        odim = OUT_DIM
    else:
        outs = (jax.ShapeDtypeStruct((2, N, ROW), jnp.float32),
                jax.ShapeDtypeStruct((N, 16), jnp.float32))
        out_specs = (pl.BlockSpec((2, BN, ROW), lambda i: (0, i, 0)),
                     pl.BlockSpec((BN, 16), lambda i: (i, 0)))
        odim = HIDDEN
    nh = 1 if heads3 else HEADS
    return pl.pallas_call(
        functools.partial(_dense_body, normalize=normalize, heads3=heads3),
        grid=(GRID,),
        in_specs=[
            pl.BlockSpec((BN, HIDDEN), lambda i: (i, 0)),
            pl.BlockSpec((2, HIDDEN), lambda i: (0, 0)),
            pl.BlockSpec((HIDDEN,), lambda i: (0,)),
            pl.BlockSpec((HIDDEN,), lambda i: (0,)),
            pl.BlockSpec((HIDDEN, odim), lambda i: (0, 0)),
            pl.BlockSpec((odim, nh), lambda i: (0, 0)),
            pl.BlockSpec((odim, nh), lambda i: (0, 0)),
        ],
        out_specs=out_specs,
        out_shape=outs,
    )(h, stats, g, be, w, al, ar)


def _combine_body(nd_ref, b_ref, erep_ref, h_ref, st_ref):
    i = pl.program_id(0)
    parts = []
    for cc in range(2):
        num = nd_ref[cc, :, 0:128]
        den = jnp.dot(nd_ref[cc, :, 132:136], erep_ref[...],
                      preferred_element_type=jnp.float32)
        ok = den > 0
        parts.append(jnp.where(ok, num / jnp.where(ok, den, 1.0), 0.0))
    hv = jnp.concatenate(parts, axis=1) + b_ref[...]
    h_ref[...] = hv

    @pl.when(i == 0)
    def _():
        st_ref[...] = jnp.zeros((2, HIDDEN), jnp.float32)

    st_ref[0, :] = st_ref[0, :] + jnp.sum(hv, axis=0)
    st_ref[1, :] = st_ref[1, :] + jnp.sum(hv * hv, axis=0)


def _combine_call(nd, b, erep):
    return pl.pallas_call(
        _combine_body,
        grid=(GRID,),
        in_specs=[
            pl.BlockSpec((2, BN, ROW), lambda i: (0, i, 0)),
            pl.BlockSpec((HIDDEN,), lambda i: (0,)),
            pl.BlockSpec((4, 128), lambda i: (0, 0)),
        ],
        out_specs=(pl.BlockSpec((BN, HIDDEN), lambda i: (i, 0)),
                   pl.BlockSpec((2, HIDDEN), lambda i: (0, 0))),
        out_shape=(jax.ShapeDtypeStruct((N, HIDDEN), jnp.float32),
                   jax.ShapeDtypeStruct((2, HIDDEN), jnp.float32)),
    )(nd, b, erep)


def _final_body(nd_ref, b_ref, out_ref):
    num = nd_ref[0, :, 0:64] + nd_ref[1, :, 0:64]
    den = nd_ref[0, :, 65:66] + nd_ref[1, :, 65:66]
    ok = den > 0
    h3 = jnp.where(ok, num / jnp.where(ok, den, 1.0), 0.0) + b_ref[...]
    m = jnp.max(h3, axis=1, keepdims=True)
    r = h3 - m
    lse = jnp.log(jnp.sum(jnp.exp(r), axis=1, keepdims=True))
    out_ref[...] = r - lse


def _final_call(nd3, b3):
    return pl.pallas_call(
        _final_body,
        grid=(GRID,),
        in_specs=[
            pl.BlockSpec((2, BN, ROW3), lambda i: (0, i, 0)),
            pl.BlockSpec((OUT_DIM,), lambda i: (0,)),
        ],
        out_specs=pl.BlockSpec((BN, OUT_DIM), lambda i: (i, 0)),
        out_shape=jax.ShapeDtypeStruct((N, OUT_DIM), jnp.float32),
    )(nd3, b3)


def _full16(v):
    return jnp.full((16,), v, jnp.int32)


def _splat(vec, lane):
    return lax.gather(
        vec, _full16(lane)[:, None],
        dimension_numbers=lax.GatherDimensionNumbers(
            offset_dims=(), collapsed_slice_dims=(0,), start_index_map=(0,)),
        slice_sizes=(1,),
        mode=lax.GatherScatterMode.PROMISE_IN_BOUNDS)


def _striped(s, fn):
    @pl.when(s < NT - 1)
    def _():
        fn(s * RPT, RPT)

    @pl.when(s == NT - 1)
    def _():
        fn((NT - 1) * RPT, RPT_L)


def _sc_edge12_body(tab, er, ei, zer, out, acc,
                    idx0, idx1, idx2, rw0, rw1, rw2, ew0, ew1, ew2,
                    semg, seme, sems):
    c = lax.axis_index("c")
    s = lax.axis_index("s")
    cn = c * N
    ch = c * HH
    idx = (idx0, idx1, idx2)
    rows = (rw0, rw1, rw2)
    errw = (ew0, ew1, ew2)
    _striped(s, lambda o, n: pltpu.sync_copy(zer.at[pl.ds(0, n)],
                                             acc.at[pl.ds(o, n)]))
    plsc.subcore_barrier()
    base = s * EPT

    def fetch(b, i):
        off = base + i * CHUNK
        pltpu.sync_copy(ei.at[c, :, pl.ds(off, CHUNK)], idx[b])
        pltpu.async_copy(tab.at[idx[b].at[0]], rows[b], semg.at[b])
        pltpu.async_copy(er.at[idx[b].at[1]], errw[b], seme.at[b])

    def compute_scatter(b):
        pltpu.make_async_copy(tab.at[idx[b].at[0]], rows[b], semg.at[b]).wait()
        pltpu.make_async_copy(er.at[idx[b].at[1]], errw[b], seme.at[b]).wait()
        for g in range(CHUNK // 16):
            e16 = jnp.arange(16, dtype=jnp.int32) + g * 16
            for h in range(HH):
                erv = plsc.load_gather(errw[b], [e16, jnp.full((16,), ch + h,
                                                               jnp.int32)])
                elv = plsc.load_gather(rows[b], [e16, _full16(128 + h)])
                sv = elv + erv
                sv = jnp.maximum(sv, 0.2 * sv)
                w = jnp.exp(sv)
                plsc.store_scatter(rows[b], [e16, _full16(132 + h)], w)
        def escale(e0, carry):
            for k in range(8):
                e = e0 * 8 + k
                wv = rows[b][e, pl.ds(128, 16)]
                for h in range(HH):
                    wsp = _splat(wv, 4 + h)
                    for j in (2 * h, 2 * h + 1):
                        sl = pl.ds(j * 16, 16)
                        rows[b][e, sl] = rows[b][e, sl] * wsp
            return carry

        lax.fori_loop(0, CHUNK // 8, escale, 0)
        pltpu.async_copy(rows[b], acc.at[idx[b].at[1]], sems.at[b], add=True)

    def wait_scatter(b):
        pltpu.make_async_copy(rows[b], acc.at[idx[b].at[1]], sems.at[b]).wait()

    fetch(0, 0)
    fetch(1, 1)

    def outer(o, carry):
        for b in range(3):
            i = 3 * o + b
            compute_scatter(b)
            b2 = (b + 2) % 3

            @pl.when(i + 2 < NCH)
            def _():
                @pl.when(i >= 1)
                def _():
                    wait_scatter(b2)
                fetch(b2, i + 2)
        return carry

    lax.fori_loop(0, NCH // 3, outer, 0)
    compute_scatter(0)
    compute_scatter(1)
    wait_scatter(2)
    wait_scatter(0)
    wait_scatter(1)
    plsc.subcore_barrier()
    _striped(s, lambda o, n: pltpu.sync_copy(acc.at[pl.ds(o, n)],
                                             out.at[pl.ds(cn + o, n)]))


def _sc_edge3_body(tab, er, ei, zer, out, acc,
                   idx0, idx1, idx2, rw0, rw1, rw2, ew0, ew1, ew2,
                   semg, seme, sems):
    c = lax.axis_index("c")
    s = lax.axis_index("s")
    idx = (idx0, idx1, idx2)
    rows = (rw0, rw1, rw2)
    errw = (ew0, ew1, ew2)
    _striped(s, lambda o, n: pltpu.sync_copy(zer.at[pl.ds(0, n)],
                                             acc.at[pl.ds(o, n)]))
    plsc.subcore_barrier()
    base = (c * NT + s) * EPT3

    def fetch(b, i):
        off = base + i * CHUNK3
        pltpu.sync_copy(ei.at[:, pl.ds(off, CHUNK3)], idx[b])
        pltpu.async_copy(tab.at[idx[b].at[0]], rows[b], semg.at[b])
        pltpu.async_copy(er.at[idx[b].at[1]], errw[b], seme.at[b])

    def compute_scatter(b):
        pltpu.make_async_copy(tab.at[idx[b].at[0]], rows[b], semg.at[b]).wait()
        pltpu.make_async_copy(er.at[idx[b].at[1]], errw[b], seme.at[b]).wait()
        for g0 in (0, 16, 24):
            e16 = jnp.arange(16, dtype=jnp.int32) + g0
            erv = plsc.load_gather(errw[b], [e16, _full16(0)])
            elv = plsc.load_gather(rows[b], [e16, _full16(64)])
            sv = elv + erv
            sv = jnp.maximum(sv, 0.2 * sv)
            w = jnp.exp(sv)
            plsc.store_scatter(rows[b], [e16, _full16(65)], w)
        def escale(e0, carry):
            for k in range(8):
                e = e0 * 8 + k
                wv = rows[b][e, pl.ds(64, 16)]
                wsp = _splat(wv, 1)
                for j in range(4):
                    sl = pl.ds(j * 16, 16)
                    rows[b][e, sl] = rows[b][e, sl] * wsp
            return carry

        lax.fori_loop(0, CHUNK3 // 8, escale, 0)
        pltpu.async_copy(rows[b], acc.at[idx[b].at[1]], sems.at[b], add=True)

    def wait_scatter(b):
        pltpu.make_async_copy(rows[b], acc.at[idx[b].at[1]], sems.at[b]).wait()

    fetch(0, 0)
    fetch(1, 1)

    def outer(o, carry):
        for b in range(3):
            i = 3 * o + b
            compute_scatter(b)
            b2 = (b + 2) % 3

            @pl.when(i + 2 < NCH3)
            def _():
                @pl.when(i >= 1)
                def _():
                    wait_scatter(b2)
                fetch(b2, i + 2)
        return carry

    lax.fori_loop(0, NCH3 // 3, outer, 0)
    compute_scatter(0)
    compute_scatter(1)
    wait_scatter(2)
    wait_scatter(0)
    wait_scatter(1)
    plsc.subcore_barrier()
    _striped(s, lambda o, n: pltpu.sync_copy(acc.at[pl.ds(o, n)],
                                             out.at[pl.ds(c * N + o, n)]))


@functools.lru_cache(maxsize=1)
def _sc_kernels():
    mesh = plsc.VectorSubcoreMesh(core_axis_name="c", subcore_axis_name="s",
                                  num_cores=2)
    cp = pltpu.CompilerParams(needs_layout_passes=False,
                              use_tc_tiling_on_sc=False)
    edge12 = pl.kernel(
        _sc_edge12_body,
        mesh=mesh,
        compiler_params=cp,
        out_type=jax.ShapeDtypeStruct((2 * N, ROW), jnp.float32),
        scratch_types=(
            [pltpu.VMEM_SHARED((N, ROW), jnp.float32)]
            + [pltpu.VMEM((2, CHUNK), jnp.int32)] * 3
            + [pltpu.VMEM((CHUNK, ROW), jnp.float32)] * 3
            + [pltpu.VMEM((CHUNK, 16), jnp.float32)] * 3
            + [pltpu.SemaphoreType.DMA((3,))] * 3
        ),
    )
    edge3 = pl.kernel(
        _sc_edge3_body,
        mesh=mesh,
        compiler_params=cp,
        out_type=jax.ShapeDtypeStruct((2 * N, ROW3), jnp.float32),
        scratch_types=(
            [pltpu.VMEM_SHARED((N, ROW3), jnp.float32)]
            + [pltpu.VMEM((2, CHUNK3), jnp.int32)] * 3
            + [pltpu.VMEM((CHUNK3, ROW3), jnp.float32)] * 3
            + [pltpu.VMEM((CHUNK3, 16), jnp.float32)] * 3
            + [pltpu.SemaphoreType.DMA((3,))] * 3
        ),
    )
    return edge12, edge3


def _head_proj(a):
    h = a.shape[0]
    eye = jnp.eye(h, dtype=jnp.float32)
    return (a[:, :, None] * eye[:, None, :]).reshape(h * a.shape[1], h)


def kernel(x, edge_index, W1, al1, ar1, b1, g1, be1,
           W2, al2, ar2, b2, g2, be2, W3, al3, ar3, b3):
    ei = edge_index.astype(jnp.int32)
    eix = jnp.stack([ei, jnp.stack([ei[0] + N, ei[1]])])

    al1p, ar1p = _head_proj(al1), _head_proj(ar1)
    al2p, ar2p = _head_proj(al2), _head_proj(ar2)
    al3p, ar3p = al3.reshape(OUT_DIM, 1), ar3.reshape(OUT_DIM, 1)
    erep = jnp.repeat(jnp.eye(4, dtype=jnp.float32), OF, axis=1)
    zer = jnp.zeros((RPT, ROW), jnp.float32)
    zer3 = jnp.zeros((RPT, ROW3), jnp.float32)
    st0 = jnp.zeros((2, HIDDEN), jnp.float32)
    gd = jnp.ones((HIDDEN,), jnp.float32)

    sc_edge12, sc_edge3 = _sc_kernels()

    tab1, er1 = _dense_call(x, st0, gd, st0[0], W1, al1p, ar1p,
                            normalize=False, heads3=False)
    nd1 = sc_edge12(tab1.reshape(2 * N, ROW), er1, eix, zer)
    h1, st1 = _combine_call(nd1.reshape(2, N, ROW), b1, erep)

    tab2, er2 = _dense_call(h1, st1, g1, be1, W2, al2p, ar2p,
                            normalize=True, heads3=False)
    nd2 = sc_edge12(tab2.reshape(2 * N, ROW), er2, eix, zer)
    h2, st2 = _combine_call(nd2.reshape(2, N, ROW), b2, erep)

    tab3, er3 = _dense_call(h2, st2, g2, be2, W3, al3p, ar3p,
                            normalize=True, heads3=True)
    nd3 = sc_edge3(tab3, er3, ei, zer3)
    return _final_call(nd3.reshape(2, N, ROW3), b3)
